# baseline (speedup 1.0000x reference)
import jax
import jax.numpy as jnp
from jax import lax
from jax.experimental import pallas as pl
from jax.experimental.pallas import tpu as pltpu
from jax.experimental.pallas import tpu_sc as plsc

EMBED_DIM = 64
SEQ_LENGTH = 200
BATCH = 4096

NUM_CORES = 2
NUM_SUBCORES = 16
NUM_WORKERS = NUM_CORES * NUM_SUBCORES
BT = BATCH // 128
UNITS = SEQ_LENGTH * BT
UNITS_PER_WORKER = UNITS // NUM_WORKERS
NBUF = 2
GROUPS = UNITS_PER_WORKER // NBUF
LANES = 16
SROW = 129


def _body(idx_hbm, word_hbm, pos_hbm, out_hbm, idx_all, pos_sh, gbuf,
          staging, *sems):
    sem_g = sems[0:NBUF]
    sem_p = sems[NBUF:2 * NBUF]
    sem_o = sems[2 * NBUF:3 * NBUF]
    c = lax.axis_index("c")
    s = lax.axis_index("s")
    wid = c * NUM_SUBCORES + s
    ubase = wid * UNITS_PER_WORKER
    lbase = c * (SEQ_LENGTH // NUM_CORES)

    pltpu.sync_copy(idx_hbm.at[pl.ds(ubase, UNITS_PER_WORKER)], idx_all)

    @pl.when(s == 0)
    def _():
        pltpu.sync_copy(
            pos_hbm.at[pl.ds(c * (SEQ_LENGTH // NUM_CORES) * EMBED_DIM,
                             (SEQ_LENGTH // NUM_CORES) * EMBED_DIM)],
            pos_sh.at[pl.ds(0, (SEQ_LENGTH // NUM_CORES) * EMBED_DIM)])

    plsc.subcore_barrier()

    def fire_gather(b, r):
        pltpu.async_copy(word_hbm.at[idx_all.at[r]], gbuf.at[b], sem_g[b])

    def fire_prefill(b, l):
        pltpu.async_copy(
            pos_sh.at[pl.ds((l - lbase) * EMBED_DIM, EMBED_DIM)],
            staging.at[b, :, pl.ds(0, 128)], sem_p[b])

    for b in range(NBUF):
        fire_gather(b, b)
        fire_prefill(b, (ubase + b) // BT)

    lane = jnp.arange(LANES, dtype=jnp.int32)

    def group_body(g, carry):
        for b in range(NBUF):
            r = g * NBUF + b
            u = ubase + r
            l = u // BT
            bt = lax.rem(u, BT)
            pltpu.make_async_copy(
                word_hbm.at[idx_all.at[r]], gbuf.at[b], sem_g[b]).wait()
            pltpu.make_async_copy(
                pos_sh.at[pl.ds((l - lbase) * EMBED_DIM, EMBED_DIM)],
                staging.at[b, :, pl.ds(0, 128)], sem_p[b]).wait()

            def t_body(t, carry2):
                col = lax.broadcast(t, (LANES,))
                for j in range(EMBED_DIM // LANES):
                    v = gbuf[b, t, pl.ds(j * LANES, LANES)]
                    plsc.addupdate_scatter(
                        staging.at[b], [lane + (j * LANES), col], v)
                return carry2

            lax.fori_loop(0, 128, t_body, 0, unroll=4)

            for dt in range(8):
                pltpu.async_copy(
                    staging.at[b, pl.ds(dt * 8, 8), pl.ds(0, 128)],
                    out_hbm.at[l, dt, bt], sem_o[b])
            @pl.when(g < GROUPS - 1)
            def _(b=b, r=r):
                for dt in range(8):
                    pltpu.make_async_copy(
                        staging.at[b, pl.ds(dt * 8, 8), pl.ds(0, 128)],
                        out_hbm.at[0, dt, 0], sem_o[b]).wait()
                fire_gather(b, r + NBUF)
                fire_prefill(b, (ubase + r + NBUF) // BT)
        return carry

    lax.fori_loop(0, GROUPS, group_body, 0)
    for b in range(NBUF):
        for dt in range(8):
            pltpu.make_async_copy(
                staging.at[b, pl.ds(dt * 8, 8), pl.ds(0, 128)],
                out_hbm.at[0, dt, 0], sem_o[b]).wait()


@jax.jit
def kernel(inputs, word_table, pos_table):
    idx = inputs.astype(jnp.int32).T.reshape(UNITS, 128)
    pos_b = jnp.broadcast_to(
        pos_table[:, :, None], (SEQ_LENGTH, EMBED_DIM, 128)).reshape(
            SEQ_LENGTH * EMBED_DIM, 128)
    mesh = plsc.VectorSubcoreMesh(
        core_axis_name="c", subcore_axis_name="s")
    run = pl.kernel(
        _body,
        out_type=jax.ShapeDtypeStruct((SEQ_LENGTH, 8, BT, 8, 128),
                                      jnp.float32),
        mesh=mesh,
        scratch_types=[
            pltpu.VMEM((UNITS_PER_WORKER, 128), jnp.int32),
            pltpu.VMEM_SHARED(
                ((SEQ_LENGTH // NUM_CORES) * EMBED_DIM + 384, 128),
                jnp.float32),
            pltpu.VMEM((NBUF, 128, EMBED_DIM), jnp.float32),
            pltpu.VMEM((NBUF, EMBED_DIM, SROW), jnp.float32),
        ] + [pltpu.SemaphoreType.DMA] * (3 * NBUF),
        compiler_params=pltpu.CompilerParams(
            use_tc_tiling_on_sc=False, needs_layout_passes=False),
    )
    out5 = run(idx, word_table, pos_b)
    return out5.transpose(2, 4, 0, 1, 3).reshape(
        BATCH, SEQ_LENGTH, EMBED_DIM)


# speedup vs baseline: 1.0071x; 1.0071x over previous
import jax
import jax.numpy as jnp
from jax import lax
from jax.experimental import pallas as pl
from jax.experimental.pallas import tpu as pltpu
from jax.experimental.pallas import tpu_sc as plsc

EMBED_DIM = 64
SEQ_LENGTH = 200
BATCH = 4096

NUM_CORES = 2
NUM_SUBCORES = 16
NUM_WORKERS = NUM_CORES * NUM_SUBCORES
SEQ_PER_WORKER = BATCH // NUM_WORKERS
HALF = SEQ_LENGTH // 2
WIDE = 2 * EMBED_DIM
NBUF = 4
GROUPS = SEQ_PER_WORKER // NBUF


def _body(idx_hbm, word_hbm, pos_hbm, out_hbm, idx_all, pos_sh, rows_v,
          *sems):
    sem_p = sems[0:NBUF]
    sem_g = sems[NBUF:2 * NBUF]
    sem_o = sems[2 * NBUF:3 * NBUF]
    c = lax.axis_index("c")
    s = lax.axis_index("s")
    wid = s * NUM_CORES + c
    base = wid * SEQ_PER_WORKER

    pltpu.sync_copy(idx_hbm.at[wid], idx_all)

    @pl.when(s == 0)
    def _():
        pltpu.sync_copy(pos_hbm, pos_sh)

    plsc.subcore_barrier()

    def group_body(g, carry):
        for b in range(NBUF):
            @pl.when(g > 0)
            def _(b=b):
                for h in range(2):
                    pltpu.make_async_copy(
                        rows_v.at[b, h],
                        out_hbm.at[pl.ds(0, HALF),
                                   pl.ds(h * EMBED_DIM, EMBED_DIM)],
                        sem_o[b]).wait()
            pltpu.async_copy(pos_sh, rows_v.at[b], sem_p[b])
        for b in range(NBUF):
            i = g * NBUF + b
            pltpu.make_async_copy(pos_sh, rows_v.at[b], sem_p[b]).wait()
            for h in range(2):
                pltpu.async_copy(
                    word_hbm.at[idx_all.at[i, h]],
                    rows_v.at[b, h], sem_g[b], add=True)
        for b in range(NBUF):
            i = g * NBUF + b
            for h in range(2):
                pltpu.make_async_copy(
                    word_hbm.at[idx_all.at[i, h]],
                    rows_v.at[b, h], sem_g[b]).wait()
            for h in range(2):
                pltpu.async_copy(
                    rows_v.at[b, h],
                    out_hbm.at[pl.ds((base + i) * HALF, HALF),
                               pl.ds(h * EMBED_DIM, EMBED_DIM)],
                    sem_o[b])
        return carry

    lax.fori_loop(0, GROUPS, group_body, 0)
    for b in range(NBUF):
        for h in range(2):
            pltpu.make_async_copy(
                rows_v.at[b, h],
                out_hbm.at[pl.ds(0, HALF),
                           pl.ds(h * EMBED_DIM, EMBED_DIM)],
                sem_o[b]).wait()


@jax.jit
def kernel(inputs, word_table, pos_table):
    idx = (inputs.astype(jnp.int32)
           .reshape(BATCH, HALF, 2)
           .transpose(0, 2, 1)
           .reshape(NUM_WORKERS, SEQ_PER_WORKER, 2, HALF))
    pos_eo = pos_table.reshape(HALF, 2, EMBED_DIM).transpose(1, 0, 2)
    mesh = plsc.VectorSubcoreMesh(
        core_axis_name="c", subcore_axis_name="s")
    run = pl.kernel(
        _body,
        out_type=jax.ShapeDtypeStruct(
            (BATCH * SEQ_LENGTH * EMBED_DIM // WIDE, WIDE), jnp.float32),
        mesh=mesh,
        scratch_types=[
            pltpu.VMEM((SEQ_PER_WORKER, 2, HALF), jnp.int32),
            pltpu.VMEM_SHARED((2, HALF, EMBED_DIM), jnp.float32),
            pltpu.VMEM((NBUF, 2, HALF, EMBED_DIM), jnp.float32),
        ] + [pltpu.SemaphoreType.DMA] * (3 * NBUF),
        compiler_params=pltpu.CompilerParams(use_tc_tiling_on_sc=False),
    )
    out128 = lax.optimization_barrier(run(idx, word_table, pos_eo))
    return out128.reshape(BATCH, SEQ_LENGTH, EMBED_DIM)
